# baseline (speedup 1.0000x reference)
import functools

import jax
import jax.numpy as jnp
from jax import lax
from jax.experimental import pallas as pl
from jax.experimental.pallas import tpu as pltpu
from jax.experimental.pallas import tpu_sc as plsc

_N_FIELDS = 26
_BATCH = 4096
_B = _BATCH * _N_FIELDS
_D = 32
_H = 16

_NC = 2
_NS = 16
_NW = _NC * _NS
_BPW = _B // _NW
_C = 128
_NCHUNK = _BPW // _C
_L = 16

_mesh = plsc.VectorSubcoreMesh(core_axis_name="c", subcore_axis_name="s")


@functools.partial(
    pl.kernel,
    mesh=_mesh,
    compiler_params=pltpu.CompilerParams(use_tc_tiling_on_sc=False),
    out_type=jax.ShapeDtypeStruct((_B, _D), jnp.float32),
    scratch_types=[
        pltpu.VMEM((_NCHUNK, _C), jnp.int32),
        pltpu.VMEM((_NCHUNK, _C), jnp.int32),
        pltpu.VMEM((_NCHUNK, _C), jnp.int32),
        pltpu.VMEM((_NCHUNK, _C), jnp.int32),
        pltpu.VMEM((_C, _H), jnp.float32),
        pltpu.VMEM((_C, _H), jnp.float32),
        pltpu.SemaphoreType.DMA,
        pltpu.SemaphoreType.DMA,
    ],
)
def _gather_kernel(table_h, ids3, off2, out_hbm,
                   ids_v, off_v, evn_v, odd_v, buf_a, buf_b, sem_a, sem_b):
    wid = lax.axis_index("s") * _NC + lax.axis_index("c")
    base = wid * _BPW
    pltpu.sync_copy(ids3.at[wid], ids_v)
    pltpu.sync_copy(off2, off_v)

    def split_ids(i, carry):
        r = i // (_C // _L)
        q = lax.rem(i, _C // _L)
        s = pl.ds(q * _L, _L)
        gid = ids_v[r, s] + off_v[r, s]
        evn = (lax.shift_left(lax.shift_right_logical(gid, 12), 13)
               + lax.shift_left(lax.bitwise_and(gid, 1023), 3)
               + lax.shift_left(lax.bitwise_and(
                   lax.shift_right_logical(gid, 10), 3), 1))
        evn_v[r, s] = evn
        odd_v[r, s] = evn + 1
        return carry

    lax.fori_loop(0, _NCHUNK * (_C // _L), split_ids, 0)

    def do_chunk(c, carry):
        cp_a = pltpu.async_copy(table_h.at[evn_v.at[c]], buf_a, sem_a)
        cp_b = pltpu.async_copy(table_h.at[odd_v.at[c]], buf_b, sem_b)
        cp_a.wait()
        cp_b.wait()
        rows = pl.ds(base + c * _C, _C)
        pltpu.sync_copy(buf_a, out_hbm.at[rows, pl.ds(0, _H)])
        pltpu.sync_copy(buf_b, out_hbm.at[rows, pl.ds(_H, _H)])
        return carry

    lax.fori_loop(0, _NCHUNK, do_chunk, 0)


_TOTAL = 2600000
_RBLK = 4096
_NBLK = (_TOTAL + _RBLK - 1) // _RBLK


def _transpose_body(in_ref, out_ref):
    x = in_ref[...]
    q = _RBLK // 4
    eye = jnp.eye(_D, dtype=jnp.float32)
    parts = []
    for u in range(4):
        xu = x[:, u * q:(u + 1) * q]
        parts.append(lax.dot_general(
            xu, eye, (((0,), (0,)), ((), ())),
            preferred_element_type=jnp.float32,
            precision=lax.Precision.HIGHEST))
    out_ref[...] = jnp.concatenate(parts, axis=1)


_tc_detranspose = pl.pallas_call(
    _transpose_body,
    grid=(_NBLK,),
    in_specs=[pl.BlockSpec((_D, _RBLK), lambda k: (0, k))],
    out_specs=pl.BlockSpec((_RBLK // 4, 128), lambda k: (k, 0)),
    out_shape=jax.ShapeDtypeStruct((_NBLK * (_RBLK // 4), 128), jnp.float32),
)


def kernel(hash_ids, weight, offsets):
    w128 = _tc_detranspose(weight.T)
    table_h = w128.reshape(-1, _H)
    ids3 = hash_ids.reshape(_NW, _NCHUNK, _C).astype(jnp.int32)
    off2 = jnp.tile(offsets.astype(jnp.int32), _BPW // _N_FIELDS).reshape(
        _NCHUNK, _C)
    out = _gather_kernel(table_h, ids3, off2)
    return out.reshape(hash_ids.shape + (weight.shape[1],))


# speedup vs baseline: 2.4857x; 2.4857x over previous
import functools

import jax
import jax.numpy as jnp
from jax import lax
from jax.experimental import pallas as pl
from jax.experimental.pallas import tpu as pltpu
from jax.experimental.pallas import tpu_sc as plsc

_N_FIELDS = 26
_BATCH = 4096
_B = _BATCH * _N_FIELDS
_D = 32
_H = 16

_NC = 2
_NS = 16
_NW = _NC * _NS
_BPW = _B // _NW
_C = 128
_NCHUNK = _BPW // _C
_L = 16

_mesh = plsc.VectorSubcoreMesh(core_axis_name="c", subcore_axis_name="s")


@functools.partial(
    pl.kernel,
    mesh=_mesh,
    compiler_params=pltpu.CompilerParams(use_tc_tiling_on_sc=False),
    out_type=jax.ShapeDtypeStruct((_B, _D), jnp.float32),
    scratch_types=[
        pltpu.VMEM((_NCHUNK, _C), jnp.int32),
        pltpu.VMEM((_NCHUNK, _C), jnp.int32),
        pltpu.VMEM((_NCHUNK, _C), jnp.int32),
        pltpu.VMEM((_NCHUNK, _C), jnp.int32),
        pltpu.VMEM((_C, _H), jnp.float32),
        pltpu.VMEM((_C, _H), jnp.float32),
        pltpu.SemaphoreType.DMA,
        pltpu.SemaphoreType.DMA,
    ],
)
def _gather_kernel(table_h, ids3, off2, out_hbm,
                   ids_v, off_v, evn_v, odd_v, buf_a, buf_b, sem_a, sem_b):
    wid = lax.axis_index("s") * _NC + lax.axis_index("c")
    base = wid * _BPW
    pltpu.sync_copy(ids3.at[wid], ids_v)
    pltpu.sync_copy(off2, off_v)

    def split_ids(i, carry):
        r = i // (_C // _L)
        q = lax.rem(i, _C // _L)
        s = pl.ds(q * _L, _L)
        gid = ids_v[r, s] + off_v[r, s]
        evn = (lax.shift_left(lax.shift_right_logical(gid, 9), 10)
               + lax.shift_left(lax.bitwise_and(gid, 127), 3)
               + lax.shift_left(lax.bitwise_and(
                   lax.shift_right_logical(gid, 7), 3), 1))
        evn_v[r, s] = evn
        odd_v[r, s] = evn + 1
        return carry

    lax.fori_loop(0, _NCHUNK * (_C // _L), split_ids, 0)

    def do_chunk(c, carry):
        cp_a = pltpu.async_copy(table_h.at[evn_v.at[c]], buf_a, sem_a)
        cp_b = pltpu.async_copy(table_h.at[odd_v.at[c]], buf_b, sem_b)
        cp_a.wait()
        cp_b.wait()
        rows = pl.ds(base + c * _C, _C)
        pltpu.sync_copy(buf_a, out_hbm.at[rows, pl.ds(0, _H)])
        pltpu.sync_copy(buf_b, out_hbm.at[rows, pl.ds(_H, _H)])
        return carry

    lax.fori_loop(0, _NCHUNK, do_chunk, 0)


_TOTAL = 2600000
_RBLK = 4096
_NBLK = (_TOTAL + _RBLK - 1) // _RBLK


def _transpose_body(in_ref, out_ref):
    x = in_ref[...]
    for w in range(_RBLK // 512):
        stacked = jnp.concatenate(
            [x[:, 512 * w + 128 * s:512 * w + 128 * (s + 1)] for s in range(4)],
            axis=0)
        out_ref[128 * w:128 * (w + 1), :] = stacked.T


_tc_detranspose = pl.pallas_call(
    _transpose_body,
    grid=(_NBLK,),
    in_specs=[pl.BlockSpec((_D, _RBLK), lambda k: (0, k))],
    out_specs=pl.BlockSpec((_RBLK // 4, 128), lambda k: (k, 0)),
    out_shape=jax.ShapeDtypeStruct((_NBLK * (_RBLK // 4), 128), jnp.float32),
)


def kernel(hash_ids, weight, offsets):
    w128 = _tc_detranspose(weight.T)
    table_h = w128.reshape(-1, _H)
    ids3 = hash_ids.reshape(_NW, _NCHUNK, _C).astype(jnp.int32)
    off2 = jnp.tile(offsets.astype(jnp.int32), _BPW // _N_FIELDS).reshape(
        _NCHUNK, _C)
    out = _gather_kernel(table_h, ids3, off2)
    return out.reshape(hash_ids.shape + (weight.shape[1],))


# speedup vs baseline: 3.2928x; 1.3247x over previous
import functools

import jax
import jax.numpy as jnp
from jax import lax
from jax.experimental import pallas as pl
from jax.experimental.pallas import tpu as pltpu
from jax.experimental.pallas import tpu_sc as plsc

_N_FIELDS = 26
_BATCH = 4096
_B = _BATCH * _N_FIELDS
_D = 32
_H = 16

_NC = 2
_NS = 16
_NW = _NC * _NS
_BPW = _B // _NW
_C = 128
_NCHUNK = _BPW // _C
_L = 16

_mesh = plsc.VectorSubcoreMesh(core_axis_name="c", subcore_axis_name="s")


@functools.partial(
    pl.kernel,
    mesh=_mesh,
    compiler_params=pltpu.CompilerParams(use_tc_tiling_on_sc=False),
    out_type=jax.ShapeDtypeStruct((_B, _D), jnp.float32),
    scratch_types=[
        pltpu.VMEM((_NCHUNK, _C), jnp.int32),
        pltpu.VMEM((_NCHUNK, _C), jnp.int32),
        pltpu.VMEM((_NCHUNK, _C), jnp.int32),
        pltpu.VMEM((_NCHUNK, _C), jnp.int32),
        pltpu.VMEM((_C, _H), jnp.float32),
        pltpu.VMEM((_C, _H), jnp.float32),
        pltpu.SemaphoreType.DMA,
        pltpu.SemaphoreType.DMA,
    ],
)
def _gather_kernel(table_h, ids3, off2, out_hbm,
                   ids_v, off_v, evn_v, odd_v, buf_a, buf_b, sem_a, sem_b):
    wid = lax.axis_index("s") * _NC + lax.axis_index("c")
    base = wid * _BPW
    pltpu.sync_copy(ids3.at[wid], ids_v)
    pltpu.sync_copy(off2, off_v)

    def split_ids(i, carry):
        r = i // (_C // _L)
        q = lax.rem(i, _C // _L)
        s = pl.ds(q * _L, _L)
        gid = ids_v[r, s] + off_v[r, s]
        evn = (lax.shift_left(lax.shift_right_logical(gid, 9), 10)
               + lax.shift_left(lax.bitwise_and(gid, 127), 3)
               + lax.shift_left(lax.bitwise_and(
                   lax.shift_right_logical(gid, 7), 3), 1))
        evn_v[r, s] = evn
        odd_v[r, s] = evn + 1
        return carry

    lax.fori_loop(0, _NCHUNK * (_C // _L), split_ids, 0)

    def do_chunk(c, carry):
        cp_a = pltpu.async_copy(table_h.at[evn_v.at[c]], buf_a, sem_a)
        cp_b = pltpu.async_copy(table_h.at[odd_v.at[c]], buf_b, sem_b)
        cp_a.wait()
        cp_b.wait()
        rows = pl.ds(base + c * _C, _C)
        pltpu.sync_copy(buf_a, out_hbm.at[rows, pl.ds(0, _H)])
        pltpu.sync_copy(buf_b, out_hbm.at[rows, pl.ds(_H, _H)])
        return carry

    lax.fori_loop(0, _NCHUNK, do_chunk, 0)


_TOTAL = 2600000
_RBLK = 8192
_NBLK = (_TOTAL + _RBLK - 1) // _RBLK


def _transpose_body(in_ref, out_ref):
    x = in_ref[...]
    for w in range(_RBLK // 512):
        stacked = jnp.concatenate(
            [x[:, 512 * w + 128 * s:512 * w + 128 * (s + 1)] for s in range(4)],
            axis=0)
        out_ref[128 * w:128 * (w + 1), :] = stacked.T


_tc_detranspose = pl.pallas_call(
    _transpose_body,
    grid=(_NBLK,),
    in_specs=[pl.BlockSpec((_D, _RBLK), lambda k: (0, k))],
    out_specs=pl.BlockSpec((_RBLK // 4, 128), lambda k: (k, 0)),
    out_shape=jax.ShapeDtypeStruct((_NBLK * (_RBLK // 4), 128), jnp.float32),
)


def kernel(hash_ids, weight, offsets):
    w128 = _tc_detranspose(weight.T)
    table_h = w128.reshape(-1, _H)
    ids3 = hash_ids.reshape(_NW, _NCHUNK, _C).astype(jnp.int32)
    off2 = jnp.tile(offsets.astype(jnp.int32), _BPW // _N_FIELDS).reshape(
        _NCHUNK, _C)
    out = _gather_kernel(table_h, ids3, off2)
    return out.reshape(hash_ids.shape + (weight.shape[1],))


# speedup vs baseline: 4.1414x; 1.2577x over previous
import functools

import jax
import jax.numpy as jnp
from jax import lax
from jax.experimental import pallas as pl
from jax.experimental.pallas import tpu as pltpu
from jax.experimental.pallas import tpu_sc as plsc

_N_FIELDS = 26
_BATCH = 4096
_B = _BATCH * _N_FIELDS
_D = 32
_H = 16

_NC = 2
_NS = 16
_NW = _NC * _NS
_BPW = _B // _NW
_C = 128
_NCHUNK = _BPW // _C
_L = 16

_mesh = plsc.VectorSubcoreMesh(core_axis_name="c", subcore_axis_name="s")


@functools.partial(
    pl.kernel,
    mesh=_mesh,
    compiler_params=pltpu.CompilerParams(use_tc_tiling_on_sc=False),
    out_type=jax.ShapeDtypeStruct((_B, _D), jnp.float32),
    scratch_types=[
        pltpu.VMEM((_NCHUNK, _C), jnp.int32),
        pltpu.VMEM((_NCHUNK, _C), jnp.int32),
        pltpu.VMEM((_NCHUNK, _C), jnp.int32),
        pltpu.VMEM((_NCHUNK, _C), jnp.int32),
        pltpu.VMEM((_C, _H), jnp.float32),
        pltpu.VMEM((_C, _H), jnp.float32),
        pltpu.SemaphoreType.DMA,
        pltpu.SemaphoreType.DMA,
    ],
)
def _gather_kernel(table_h, ids3, off2, out_hbm,
                   ids_v, off_v, evn_v, odd_v, buf_a, buf_b, sem_a, sem_b):
    wid = lax.axis_index("s") * _NC + lax.axis_index("c")
    base = wid * _BPW
    pltpu.sync_copy(ids3.at[wid], ids_v)
    pltpu.sync_copy(off2, off_v)

    def split_ids(i, carry):
        r = i // (_C // _L)
        q = lax.rem(i, _C // _L)
        s = pl.ds(q * _L, _L)
        gid = ids_v[r, s] + off_v[r, s]
        evn = (lax.shift_left(lax.shift_right_logical(gid, 9), 10)
               + lax.shift_left(lax.bitwise_and(gid, 127), 3)
               + lax.shift_left(lax.bitwise_and(
                   lax.shift_right_logical(gid, 7), 3), 1))
        evn_v[r, s] = evn
        odd_v[r, s] = evn + 1
        return carry

    lax.fori_loop(0, _NCHUNK * (_C // _L), split_ids, 0)

    def do_chunk(c, carry):
        cp_a = pltpu.async_copy(table_h.at[evn_v.at[c]], buf_a, sem_a)
        cp_b = pltpu.async_copy(table_h.at[odd_v.at[c]], buf_b, sem_b)
        cp_a.wait()
        cp_b.wait()
        rows = pl.ds(base + c * _C, _C)
        pltpu.sync_copy(buf_a, out_hbm.at[rows, pl.ds(0, _H)])
        pltpu.sync_copy(buf_b, out_hbm.at[rows, pl.ds(_H, _H)])
        return carry

    lax.fori_loop(0, _NCHUNK, do_chunk, 0)


_TOTAL = 2600000
_RBLK = 16384
_NBLK = (_TOTAL + _RBLK - 1) // _RBLK


def _transpose_body(in_ref, out_ref):
    x = in_ref[...]
    for w in range(_RBLK // 512):
        stacked = jnp.concatenate(
            [x[:, 512 * w + 128 * s:512 * w + 128 * (s + 1)] for s in range(4)],
            axis=0)
        out_ref[128 * w:128 * (w + 1), :] = stacked.T


_tc_detranspose = pl.pallas_call(
    _transpose_body,
    grid=(_NBLK,),
    in_specs=[pl.BlockSpec((_D, _RBLK), lambda k: (0, k))],
    out_specs=pl.BlockSpec((_RBLK // 4, 128), lambda k: (k, 0)),
    out_shape=jax.ShapeDtypeStruct((_NBLK * (_RBLK // 4), 128), jnp.float32),
)


def kernel(hash_ids, weight, offsets):
    w128 = _tc_detranspose(weight.T)
    table_h = w128.reshape(-1, _H)
    ids3 = hash_ids.reshape(_NW, _NCHUNK, _C).astype(jnp.int32)
    off2 = jnp.tile(offsets.astype(jnp.int32), _BPW // _N_FIELDS).reshape(
        _NCHUNK, _C)
    out = _gather_kernel(table_h, ids3, off2)
    return out.reshape(hash_ids.shape + (weight.shape[1],))


# speedup vs baseline: 4.5946x; 1.1094x over previous
import functools

import jax
import jax.numpy as jnp
from jax import lax
from jax.experimental import pallas as pl
from jax.experimental.pallas import tpu as pltpu
from jax.experimental.pallas import tpu_sc as plsc

_N_FIELDS = 26
_BATCH = 4096
_B = _BATCH * _N_FIELDS
_D = 32
_H = 16

_NC = 2
_NS = 16
_NW = _NC * _NS
_BPW = _B // _NW
_C = 128
_NCHUNK = _BPW // _C
_L = 16

_mesh = plsc.VectorSubcoreMesh(core_axis_name="c", subcore_axis_name="s")


@functools.partial(
    pl.kernel,
    mesh=_mesh,
    compiler_params=pltpu.CompilerParams(use_tc_tiling_on_sc=False),
    out_type=jax.ShapeDtypeStruct((_B, _D), jnp.float32),
    scratch_types=[
        pltpu.VMEM((_NCHUNK, _C), jnp.int32),
        pltpu.VMEM((_NCHUNK, _C), jnp.int32),
        pltpu.VMEM((_NCHUNK, _C), jnp.int32),
        pltpu.VMEM((_NCHUNK, _C), jnp.int32),
        pltpu.VMEM((_C, _H), jnp.float32),
        pltpu.VMEM((_C, _H), jnp.float32),
        pltpu.SemaphoreType.DMA,
        pltpu.SemaphoreType.DMA,
    ],
)
def _gather_kernel(table_h, ids3, off2, out_hbm,
                   ids_v, off_v, evn_v, odd_v, buf_a, buf_b, sem_a, sem_b):
    wid = lax.axis_index("s") * _NC + lax.axis_index("c")
    base = wid * _BPW
    pltpu.sync_copy(ids3.at[wid], ids_v)
    pltpu.sync_copy(off2, off_v)

    def split_ids(i, carry):
        r = i // (_C // _L)
        q = lax.rem(i, _C // _L)
        s = pl.ds(q * _L, _L)
        gid = ids_v[r, s] + off_v[r, s]
        evn = (lax.shift_left(lax.shift_right_logical(gid, 9), 10)
               + lax.shift_left(lax.bitwise_and(gid, 127), 3)
               + lax.shift_left(lax.bitwise_and(
                   lax.shift_right_logical(gid, 7), 3), 1))
        evn_v[r, s] = evn
        odd_v[r, s] = evn + 1
        return carry

    lax.fori_loop(0, _NCHUNK * (_C // _L), split_ids, 0)

    def do_chunk(c, carry):
        cp_a = pltpu.async_copy(table_h.at[evn_v.at[c]], buf_a, sem_a)
        cp_b = pltpu.async_copy(table_h.at[odd_v.at[c]], buf_b, sem_b)
        cp_a.wait()
        cp_b.wait()
        rows = pl.ds(base + c * _C, _C)
        pltpu.sync_copy(buf_a, out_hbm.at[rows, pl.ds(0, _H)])
        pltpu.sync_copy(buf_b, out_hbm.at[rows, pl.ds(_H, _H)])
        return carry

    lax.fori_loop(0, _NCHUNK, do_chunk, 0)


_TOTAL = 2600000
_RBLK = 32768
_NBLK = (_TOTAL + _RBLK - 1) // _RBLK


def _transpose_body(in_ref, out_ref):
    x = in_ref[...]
    for w in range(_RBLK // 512):
        stacked = jnp.concatenate(
            [x[:, 512 * w + 128 * s:512 * w + 128 * (s + 1)] for s in range(4)],
            axis=0)
        out_ref[128 * w:128 * (w + 1), :] = stacked.T


_tc_detranspose = pl.pallas_call(
    _transpose_body,
    grid=(_NBLK,),
    in_specs=[pl.BlockSpec((_D, _RBLK), lambda k: (0, k))],
    out_specs=pl.BlockSpec((_RBLK // 4, 128), lambda k: (k, 0)),
    out_shape=jax.ShapeDtypeStruct((_NBLK * (_RBLK // 4), 128), jnp.float32),
)


def kernel(hash_ids, weight, offsets):
    w128 = _tc_detranspose(weight.T)
    table_h = w128.reshape(-1, _H)
    ids3 = hash_ids.reshape(_NW, _NCHUNK, _C).astype(jnp.int32)
    off2 = jnp.tile(offsets.astype(jnp.int32), _BPW // _N_FIELDS).reshape(
        _NCHUNK, _C)
    out = _gather_kernel(table_h, ids3, off2)
    return out.reshape(hash_ids.shape + (weight.shape[1],))


# speedup vs baseline: 4.6587x; 1.0139x over previous
import functools

import jax
import jax.numpy as jnp
from jax import lax
from jax.experimental import pallas as pl
from jax.experimental.pallas import tpu as pltpu
from jax.experimental.pallas import tpu_sc as plsc

_N_FIELDS = 26
_BATCH = 4096
_B = _BATCH * _N_FIELDS
_D = 32
_H = 16

_NC = 2
_NS = 16
_NW = _NC * _NS
_BPW = _B // _NW
_C = 128
_NCHUNK = _BPW // _C
_L = 16

_mesh = plsc.VectorSubcoreMesh(core_axis_name="c", subcore_axis_name="s")


@functools.partial(
    pl.kernel,
    mesh=_mesh,
    compiler_params=pltpu.CompilerParams(use_tc_tiling_on_sc=False),
    out_type=jax.ShapeDtypeStruct((_B, _D), jnp.float32),
    scratch_types=[
        pltpu.VMEM((_NCHUNK, _C), jnp.int32),
        pltpu.VMEM((_NCHUNK, _C), jnp.int32),
        pltpu.VMEM((_NCHUNK, _C), jnp.int32),
        pltpu.VMEM((_NCHUNK, _C), jnp.int32),
        pltpu.VMEM((_C, _H), jnp.float32),
        pltpu.VMEM((_C, _H), jnp.float32),
        pltpu.SemaphoreType.DMA,
        pltpu.SemaphoreType.DMA,
    ],
)
def _gather_kernel(table_h, ids3, off2, out_hbm,
                   ids_v, off_v, evn_v, odd_v, buf_a, buf_b, sem_a, sem_b):
    wid = lax.axis_index("s") * _NC + lax.axis_index("c")
    base = wid * _BPW
    pltpu.sync_copy(ids3.at[wid], ids_v)
    pltpu.sync_copy(off2, off_v)

    def split_ids(i, carry):
        r = i // (_C // _L)
        q = lax.rem(i, _C // _L)
        s = pl.ds(q * _L, _L)
        gid = ids_v[r, s] + off_v[r, s]
        evn = (lax.shift_left(lax.shift_right_logical(gid, 9), 10)
               + lax.shift_left(lax.bitwise_and(gid, 127), 3)
               + lax.shift_left(lax.bitwise_and(
                   lax.shift_right_logical(gid, 7), 3), 1))
        evn_v[r, s] = evn
        odd_v[r, s] = evn + 1
        return carry

    lax.fori_loop(0, _NCHUNK * (_C // _L), split_ids, 0)

    def do_chunk(c, carry):
        cp_a = pltpu.async_copy(table_h.at[evn_v.at[c]], buf_a, sem_a)
        cp_b = pltpu.async_copy(table_h.at[odd_v.at[c]], buf_b, sem_b)
        cp_a.wait()
        cp_b.wait()
        rows = pl.ds(base + c * _C, _C)
        pltpu.sync_copy(buf_a, out_hbm.at[rows, pl.ds(0, _H)])
        pltpu.sync_copy(buf_b, out_hbm.at[rows, pl.ds(_H, _H)])
        return carry

    lax.fori_loop(0, _NCHUNK, do_chunk, 0)


_TOTAL = 2600000
_RBLK = 65536
_NBLK = (_TOTAL + _RBLK - 1) // _RBLK


def _transpose_body(in_ref, out_ref):
    x = in_ref[...]
    for w in range(_RBLK // 512):
        stacked = jnp.concatenate(
            [x[:, 512 * w + 128 * s:512 * w + 128 * (s + 1)] for s in range(4)],
            axis=0)
        out_ref[128 * w:128 * (w + 1), :] = stacked.T


_tc_detranspose = pl.pallas_call(
    _transpose_body,
    grid=(_NBLK,),
    in_specs=[pl.BlockSpec((_D, _RBLK), lambda k: (0, k))],
    out_specs=pl.BlockSpec((_RBLK // 4, 128), lambda k: (k, 0)),
    out_shape=jax.ShapeDtypeStruct((_NBLK * (_RBLK // 4), 128), jnp.float32),
)


def kernel(hash_ids, weight, offsets):
    w128 = _tc_detranspose(weight.T)
    table_h = w128.reshape(-1, _H)
    ids3 = hash_ids.reshape(_NW, _NCHUNK, _C).astype(jnp.int32)
    off2 = jnp.tile(offsets.astype(jnp.int32), _BPW // _N_FIELDS).reshape(
        _NCHUNK, _C)
    out = _gather_kernel(table_h, ids3, off2)
    return out.reshape(hash_ids.shape + (weight.shape[1],))


# speedup vs baseline: 4.6598x; 1.0003x over previous
import functools

import jax
import jax.numpy as jnp
from jax import lax
from jax.experimental import pallas as pl
from jax.experimental.pallas import tpu as pltpu
from jax.experimental.pallas import tpu_sc as plsc

_N_FIELDS = 26
_BATCH = 4096
_B = _BATCH * _N_FIELDS
_D = 32
_H = 16

_NC = 2
_NS = 16
_NW = _NC * _NS
_BPW = _B // _NW
_C = 128
_NCHUNK = _BPW // _C
_L = 16

_mesh = plsc.VectorSubcoreMesh(core_axis_name="c", subcore_axis_name="s")


@functools.partial(
    pl.kernel,
    mesh=_mesh,
    compiler_params=pltpu.CompilerParams(use_tc_tiling_on_sc=False),
    out_type=jax.ShapeDtypeStruct((_B, _D), jnp.float32),
    scratch_types=[
        pltpu.VMEM((_NCHUNK, _C), jnp.int32),
        pltpu.VMEM((_NCHUNK, _C), jnp.int32),
        pltpu.VMEM((_NCHUNK, _C), jnp.int32),
        pltpu.VMEM((_NCHUNK, _C), jnp.int32),
        pltpu.VMEM((_C, _H), jnp.float32),
        pltpu.VMEM((_C, _H), jnp.float32),
        pltpu.SemaphoreType.DMA,
        pltpu.SemaphoreType.DMA,
    ],
)
def _gather_kernel(table_h, ids3, off2, out_hbm,
                   ids_v, off_v, evn_v, odd_v, buf_a, buf_b, sem_a, sem_b):
    wid = lax.axis_index("s") * _NC + lax.axis_index("c")
    base = wid * _BPW
    pltpu.sync_copy(ids3.at[wid], ids_v)
    pltpu.sync_copy(off2, off_v)

    def split_ids(i, carry):
        r = i // (_C // _L)
        q = lax.rem(i, _C // _L)
        s = pl.ds(q * _L, _L)
        gid = ids_v[r, s] + off_v[r, s]
        evn = (lax.shift_left(lax.shift_right_logical(gid, 9), 10)
               + lax.shift_left(lax.bitwise_and(gid, 127), 3)
               + lax.shift_left(lax.bitwise_and(
                   lax.shift_right_logical(gid, 7), 3), 1))
        evn_v[r, s] = evn
        odd_v[r, s] = evn + 1
        return carry

    lax.fori_loop(0, _NCHUNK * (_C // _L), split_ids, 0)

    def do_chunk(c, carry):
        cp_a = pltpu.async_copy(table_h.at[evn_v.at[c]], buf_a, sem_a)
        cp_b = pltpu.async_copy(table_h.at[odd_v.at[c]], buf_b, sem_b)
        cp_a.wait()
        cp_b.wait()
        rows = pl.ds(base + c * _C, _C)
        pltpu.sync_copy(buf_a, out_hbm.at[rows, pl.ds(0, _H)])
        pltpu.sync_copy(buf_b, out_hbm.at[rows, pl.ds(_H, _H)])
        return carry

    lax.fori_loop(0, _NCHUNK, do_chunk, 0)


_TOTAL = 2600000
_RBLK = 98304
_NBLK = (_TOTAL + _RBLK - 1) // _RBLK


def _transpose_body(in_ref, out_ref):
    x = in_ref[...]
    for w in range(_RBLK // 512):
        stacked = jnp.concatenate(
            [x[:, 512 * w + 128 * s:512 * w + 128 * (s + 1)] for s in range(4)],
            axis=0)
        out_ref[128 * w:128 * (w + 1), :] = stacked.T


_tc_detranspose = pl.pallas_call(
    _transpose_body,
    grid=(_NBLK,),
    in_specs=[pl.BlockSpec((_D, _RBLK), lambda k: (0, k))],
    out_specs=pl.BlockSpec((_RBLK // 4, 128), lambda k: (k, 0)),
    out_shape=jax.ShapeDtypeStruct((_NBLK * (_RBLK // 4), 128), jnp.float32),
)


def kernel(hash_ids, weight, offsets):
    w128 = _tc_detranspose(weight.T)
    table_h = w128.reshape(-1, _H)
    ids3 = hash_ids.reshape(_NW, _NCHUNK, _C).astype(jnp.int32)
    off2 = jnp.tile(offsets.astype(jnp.int32), _BPW // _N_FIELDS).reshape(
        _NCHUNK, _C)
    out = _gather_kernel(table_h, ids3, off2)
    return out.reshape(hash_ids.shape + (weight.shape[1],))


# speedup vs baseline: 4.7747x; 1.0247x over previous
import functools

import jax
import jax.numpy as jnp
from jax import lax
from jax.experimental import pallas as pl
from jax.experimental.pallas import tpu as pltpu
from jax.experimental.pallas import tpu_sc as plsc

_N_FIELDS = 26
_BATCH = 4096
_B = _BATCH * _N_FIELDS
_D = 32
_H = 16

_NC = 2
_NS = 16
_NW = _NC * _NS
_BPW = _B // _NW
_C = 128
_NCHUNK = _BPW // _C
_L = 16

_mesh = plsc.VectorSubcoreMesh(core_axis_name="c", subcore_axis_name="s")


@functools.partial(
    pl.kernel,
    mesh=_mesh,
    compiler_params=pltpu.CompilerParams(use_tc_tiling_on_sc=False),
    out_type=jax.ShapeDtypeStruct((_B, _D), jnp.float32),
    scratch_types=[
        pltpu.VMEM((_NCHUNK, _C), jnp.int32),
        pltpu.VMEM((_NCHUNK, _C), jnp.int32),
        pltpu.VMEM((_NCHUNK, _C), jnp.int32),
        pltpu.VMEM((_NCHUNK, _C), jnp.int32),
        pltpu.VMEM((_C, _H), jnp.float32),
        pltpu.VMEM((_C, _H), jnp.float32),
        pltpu.VMEM((_C, _H), jnp.float32),
        pltpu.VMEM((_C, _H), jnp.float32),
        pltpu.SemaphoreType.DMA,
        pltpu.SemaphoreType.DMA,
        pltpu.SemaphoreType.DMA,
        pltpu.SemaphoreType.DMA,
    ],
)
def _gather_kernel(table_h, ids3, off2, out_hbm,
                   ids_v, off_v, evn_v, odd_v,
                   buf_a0, buf_b0, buf_a1, buf_b1,
                   sem_a0, sem_b0, sem_a1, sem_b1):
    wid = lax.axis_index("s") * _NC + lax.axis_index("c")
    base = wid * _BPW
    pltpu.sync_copy(ids3.at[wid], ids_v)
    pltpu.sync_copy(off2, off_v)

    def split_ids(i, carry):
        r = i // (_C // _L)
        q = lax.rem(i, _C // _L)
        s = pl.ds(q * _L, _L)
        gid = ids_v[r, s] + off_v[r, s]
        evn = (lax.shift_left(lax.shift_right_logical(gid, 9), 10)
               + lax.shift_left(lax.bitwise_and(gid, 127), 3)
               + lax.shift_left(lax.bitwise_and(
                   lax.shift_right_logical(gid, 7), 3), 1))
        evn_v[r, s] = evn
        odd_v[r, s] = evn + 1
        return carry

    lax.fori_loop(0, _NCHUNK * (_C // _L), split_ids, 0)

    bufs = [(buf_a0, buf_b0, sem_a0, sem_b0), (buf_a1, buf_b1, sem_a1, sem_b1)]

    def issue(c):
        ba, bb, sa, sb = bufs[c % 2]
        cp_a = pltpu.async_copy(table_h.at[evn_v.at[c]], ba, sa)
        cp_b = pltpu.async_copy(table_h.at[odd_v.at[c]], bb, sb)
        return cp_a, cp_b

    inflight = issue(0)
    for c in range(_NCHUNK):
        cp_a, cp_b = inflight
        cp_a.wait()
        cp_b.wait()
        if c + 1 < _NCHUNK:
            inflight = issue(c + 1)
        ba, bb, _, _ = bufs[c % 2]
        rows = pl.ds(base + c * _C, _C)
        pltpu.sync_copy(ba, out_hbm.at[rows, pl.ds(0, _H)])
        pltpu.sync_copy(bb, out_hbm.at[rows, pl.ds(_H, _H)])


_TOTAL = 2600000
_RBLK = 98304
_NBLK = (_TOTAL + _RBLK - 1) // _RBLK


def _transpose_body(in_ref, out_ref):
    x = in_ref[...]
    for w in range(_RBLK // 512):
        stacked = jnp.concatenate(
            [x[:, 512 * w + 128 * s:512 * w + 128 * (s + 1)] for s in range(4)],
            axis=0)
        out_ref[128 * w:128 * (w + 1), :] = stacked.T


_tc_detranspose = pl.pallas_call(
    _transpose_body,
    grid=(_NBLK,),
    in_specs=[pl.BlockSpec((_D, _RBLK), lambda k: (0, k))],
    out_specs=pl.BlockSpec((_RBLK // 4, 128), lambda k: (k, 0)),
    out_shape=jax.ShapeDtypeStruct((_NBLK * (_RBLK // 4), 128), jnp.float32),
)


def kernel(hash_ids, weight, offsets):
    w128 = _tc_detranspose(weight.T)
    table_h = w128.reshape(-1, _H)
    ids3 = hash_ids.reshape(_NW, _NCHUNK, _C).astype(jnp.int32)
    off2 = jnp.tile(offsets.astype(jnp.int32), _BPW // _N_FIELDS).reshape(
        _NCHUNK, _C)
    out = _gather_kernel(table_h, ids3, off2)
    return out.reshape(hash_ids.shape + (weight.shape[1],))
